# baseline (speedup 1.0000x reference)
import functools

import jax
import jax.numpy as jnp
from jax import lax
from jax.experimental import pallas as pl
from jax.experimental.pallas import tpu as pltpu
from jax.experimental.pallas import tpu_sc as plsc

N = 1024
NBINS = 64
NMASK = 38
K = 25
RB = 8
BIG = 1.0e6
L = 16
NW = 32
RPW = N // NW


def _ent_body(lg_ref, rit_ref, ri_ref, cit_ref, ci_ref, out_ref):
    x = lg_ref[...]

    xc = jnp.clip(x, -60.0, 60.0)
    e = jnp.exp(xc)
    sall = jnp.sum(e, axis=1)
    s0 = jnp.sum(e[:, :NMASK, :], axis=1)
    s1 = jnp.sum(xc[:, :NMASK, :] * e[:, :NMASK, :], axis=1)

    ent = jnp.log(sall) - s1 / s0

    ri_rows = rit_ref[...]
    ci_rows = cit_ref[...]
    cond = (jnp.abs(ri_rows - ri_ref[...]) >= 10) | (ci_rows != ci_ref[...])
    out_ref[...] = jnp.where(cond, ent, BIG)


def _sort16(x):
    return plsc.sort_key_val(x, x)[0]


def _rv(x):
    return lax.rev(x, (0,))


def _topk_sc_body(ent_hbm, out_hbm, rows_v, res_v, cmin_v):
    wid = lax.axis_index("s") * 2 + lax.axis_index("c")
    base = wid * RPW
    pltpu.sync_copy(ent_hbm.at[pl.ds(base, RPW)], rows_v)

    lane = lax.iota(jnp.int32, L)
    inf16 = jnp.full((L,), jnp.inf, jnp.float32)
    NCH = N // L

    def chunk_fn(r, c, carry):
        a, b, bound = carry
        cmin = cmin_v[c]

        def do_merge(_):
            v = rows_v[r, pl.ds(c * L, L)]
            sc = _sort16(v)
            lo1 = _sort16(jnp.minimum(b, _rv(sc)))
            hi1 = _sort16(jnp.maximum(b, _rv(sc)))
            na = _sort16(jnp.minimum(a, _rv(lo1)))
            mid = _sort16(jnp.maximum(a, _rv(lo1)))
            nb = _sort16(jnp.minimum(mid, _rv(hi1)))
            return na, nb, jnp.max(nb)

        return lax.cond(cmin < bound, do_merge, lambda _: carry, 0)

    def row_fn(g, j, acc):
        r = g * L + j

        def premin(c, _):
            cmin_v[c] = jnp.min(rows_v[r, pl.ds(c * L, L)])
            return 0

        lax.fori_loop(0, NCH, premin, 0)

        a, b, _ = lax.fori_loop(0, NCH, functools.partial(chunk_fn, r),
                                (inf16, inf16, jnp.float32(jnp.inf)))
        contrib = a + jnp.where(lane < K - L, b, 0.0)
        s25 = jnp.sum(contrib)
        return jnp.where(lane == j, s25, acc)

    for g in range(RPW // L):
        acc = lax.fori_loop(0, L, functools.partial(row_fn, g),
                            jnp.zeros((L,), jnp.float32))
        res_v[pl.ds(g * L, L)] = acc

    pltpu.sync_copy(res_v, out_hbm.at[pl.ds(base, RPW)])


def kernel(distogram_logits, residue_index, asym_id):
    lg = jnp.transpose(distogram_logits, (0, 1, 3, 2)).reshape(N, NBINS, N)
    ri = residue_index.reshape(1, N).astype(jnp.int32)
    ci = asym_id.reshape(1, N).astype(jnp.int32)
    rit = ri.reshape(N, 1)
    cit = ci.reshape(N, 1)

    ent = pl.pallas_call(
        _ent_body,
        grid=(N // RB,),
        in_specs=[
            pl.BlockSpec((RB, NBINS, N), lambda i: (i, 0, 0)),
            pl.BlockSpec((RB, 1), lambda i: (i, 0)),
            pl.BlockSpec((1, N), lambda i: (0, 0)),
            pl.BlockSpec((RB, 1), lambda i: (i, 0)),
            pl.BlockSpec((1, N), lambda i: (0, 0)),
        ],
        out_specs=pl.BlockSpec((RB, N), lambda i: (i, 0)),
        out_shape=jax.ShapeDtypeStruct((N, N), jnp.float32),
    )(lg, rit, ri, cit, ci)

    mesh = plsc.VectorSubcoreMesh(core_axis_name="c", subcore_axis_name="s")
    topk = functools.partial(
        pl.kernel,
        mesh=mesh,
        out_type=jax.ShapeDtypeStruct((N,), jnp.float32),
        scratch_types=[
            pltpu.VMEM((RPW, N), jnp.float32),
            pltpu.VMEM((RPW,), jnp.float32),
            pltpu.SMEM((N // L,), jnp.float32),
        ],
        compiler_params=pltpu.CompilerParams(needs_layout_passes=False),
    )(_topk_sc_body)
    row_sums = topk(ent)

    return jnp.sum(row_sums) * (1.0 / (K * N))


# speedup vs baseline: 1.2603x; 1.2603x over previous
import functools

import jax
import jax.numpy as jnp
from jax import lax
from jax.experimental import pallas as pl
from jax.experimental.pallas import tpu as pltpu
from jax.experimental.pallas import tpu_sc as plsc

N = 1024
NBINS = 64
NMASK = 38
K = 25
RB = 8
TB = 256
BIG = 1.0e6
L = 16
NW = 32
SCROWS = N // 2
RPW = SCROWS // NW


def _ent_body(lg_ref, rit_ref, ri_ref, cit_ref, ci_ref, out_ref):
    x = lg_ref[...]

    xc = jnp.clip(x, -60.0, 60.0)
    e = jnp.exp(xc)
    sall = jnp.sum(e, axis=1)
    s0 = jnp.sum(e[:, :NMASK, :], axis=1)
    s1 = jnp.sum(xc[:, :NMASK, :] * e[:, :NMASK, :], axis=1)

    ent = jnp.log(sall) - s1 / s0

    ri_rows = rit_ref[...]
    ci_rows = cit_ref[...]
    cond = (jnp.abs(ri_rows - ri_ref[...]) >= 10) | (ci_rows != ci_ref[...])
    out_ref[...] = jnp.where(cond, ent, BIG)


def _topk_tc_body(ent_ref, out_ref):
    i = pl.program_id(0)
    work = ent_ref[...]
    colio = jax.lax.broadcasted_iota(jnp.int32, (TB, N), 1)
    acc = jnp.zeros((TB, 1), jnp.float32)
    for _ in range(K):
        m = jnp.min(work, axis=1, keepdims=True)
        acc = acc + m
        ismin = work == m
        first = jnp.min(jnp.where(ismin, colio, N), axis=1, keepdims=True)
        work = jnp.where(colio == first, jnp.float32(jnp.inf), work)

    partial = jnp.sum(acc) * (1.0 / (K * N))

    @pl.when(i == 0)
    def _():
        out_ref[...] = jnp.zeros_like(out_ref)

    out_ref[...] += jnp.reshape(partial, (1, 1))


def _sort16(x):
    return plsc.sort_key_val(x, x)[0]


def _rv(x):
    return lax.rev(x, (0,))


def _topk_sc_body(ent_hbm, out_hbm, rows_v, res_v, cmin_v):
    wid = lax.axis_index("s") * 2 + lax.axis_index("c")
    base = wid * RPW
    pltpu.sync_copy(ent_hbm.at[pl.ds(SCROWS + base, RPW)], rows_v)

    lane = lax.iota(jnp.int32, L)
    inf16 = jnp.full((L,), jnp.inf, jnp.float32)
    NCH = N // L

    def chunk_fn(r, c, carry):
        a, b, bound = carry
        cmin = cmin_v[c]

        def do_merge(_):
            v = rows_v[r, pl.ds(c * L, L)]
            sc = _sort16(v)
            lo1 = _sort16(jnp.minimum(b, _rv(sc)))
            hi1 = _sort16(jnp.maximum(b, _rv(sc)))
            na = _sort16(jnp.minimum(a, _rv(lo1)))
            mid = _sort16(jnp.maximum(a, _rv(lo1)))
            nb = _sort16(jnp.minimum(mid, _rv(hi1)))
            return na, nb, jnp.max(nb)

        return lax.cond(cmin < bound, do_merge, lambda _: carry, 0)

    def row_fn(j, acc):
        r = j

        def premin(c, _):
            cmin_v[c] = jnp.min(rows_v[r, pl.ds(c * L, L)])
            return 0

        lax.fori_loop(0, NCH, premin, 0)

        a, b, _ = lax.fori_loop(0, NCH, functools.partial(chunk_fn, r),
                                (inf16, inf16, jnp.float32(jnp.inf)))
        contrib = a + jnp.where(lane < K - L, b, 0.0)
        s25 = jnp.sum(contrib)
        return jnp.where(lane == j, s25, acc)

    acc = lax.fori_loop(0, RPW, row_fn, jnp.zeros((L,), jnp.float32))
    res_v[pl.ds(0, L)] = acc

    pltpu.sync_copy(res_v, out_hbm.at[pl.ds(base, RPW)])


def kernel(distogram_logits, residue_index, asym_id):
    lg = jnp.transpose(distogram_logits, (0, 1, 3, 2)).reshape(N, NBINS, N)
    ri = residue_index.reshape(1, N).astype(jnp.int32)
    ci = asym_id.reshape(1, N).astype(jnp.int32)
    rit = ri.reshape(N, 1)
    cit = ci.reshape(N, 1)

    ent = pl.pallas_call(
        _ent_body,
        grid=(N // RB,),
        in_specs=[
            pl.BlockSpec((RB, NBINS, N), lambda i: (i, 0, 0)),
            pl.BlockSpec((RB, 1), lambda i: (i, 0)),
            pl.BlockSpec((1, N), lambda i: (0, 0)),
            pl.BlockSpec((RB, 1), lambda i: (i, 0)),
            pl.BlockSpec((1, N), lambda i: (0, 0)),
        ],
        out_specs=pl.BlockSpec((RB, N), lambda i: (i, 0)),
        out_shape=jax.ShapeDtypeStruct((N, N), jnp.float32),
    )(lg, rit, ri, cit, ci)

    mesh = plsc.VectorSubcoreMesh(core_axis_name="c", subcore_axis_name="s")
    topk_sc = functools.partial(
        pl.kernel,
        mesh=mesh,
        out_type=jax.ShapeDtypeStruct((SCROWS,), jnp.float32),
        scratch_types=[
            pltpu.VMEM((RPW, N), jnp.float32),
            pltpu.VMEM((RPW,), jnp.float32),
            pltpu.SMEM((N // L,), jnp.float32),
        ],
        compiler_params=pltpu.CompilerParams(needs_layout_passes=False),
    )(_topk_sc_body)
    sc_row_sums = topk_sc(ent)

    tc_part = pl.pallas_call(
        _topk_tc_body,
        grid=(SCROWS // TB,),
        in_specs=[pl.BlockSpec((TB, N), lambda i: (i, 0))],
        out_specs=pl.BlockSpec((1, 1), lambda i: (0, 0)),
        out_shape=jax.ShapeDtypeStruct((1, 1), jnp.float32),
    )(ent)

    return tc_part[0, 0] + jnp.sum(sc_row_sums) * (1.0 / (K * N))


# speedup vs baseline: 1.4458x; 1.1472x over previous
import functools

import jax
import jax.numpy as jnp
from jax import lax
from jax.experimental import pallas as pl
from jax.experimental.pallas import tpu as pltpu
from jax.experimental.pallas import tpu_sc as plsc

N = 1024
NBINS = 64
NMASK = 38
K = 25
RB = 8
TB = 256
BIG = 1.0e6
L = 16
NW = 32
SCROWS = 256
RPW = SCROWS // NW


def _ent_body(lg_ref, rit_ref, ri_ref, cit_ref, ci_ref, out_ref):
    x = lg_ref[...]

    xc = jnp.clip(x, -60.0, 60.0)
    e = jnp.exp(xc)
    sall = jnp.sum(e, axis=1)
    s0 = jnp.sum(e[:, :NMASK, :], axis=1)
    s1 = jnp.sum(xc[:, :NMASK, :] * e[:, :NMASK, :], axis=1)

    ent = jnp.log(sall) - s1 / s0

    ri_rows = rit_ref[...]
    ci_rows = cit_ref[...]
    cond = (jnp.abs(ri_rows - ri_ref[...]) >= 10) | (ci_rows != ci_ref[...])
    out_ref[...] = jnp.where(cond, ent, BIG)


def _topk_tc_body(ent_ref, out_ref):
    i = pl.program_id(0)
    work = ent_ref[...]
    colio = jax.lax.broadcasted_iota(jnp.int32, (TB, N), 1)
    acc = jnp.zeros((TB, 1), jnp.float32)
    for _ in range(K):
        m = jnp.min(work, axis=1, keepdims=True)
        acc = acc + m
        ismin = work == m
        first = jnp.min(jnp.where(ismin, colio, N), axis=1, keepdims=True)
        work = jnp.where(colio == first, jnp.float32(jnp.inf), work)

    partial = jnp.sum(acc) * (1.0 / (K * N))

    @pl.when(i == 0)
    def _():
        out_ref[...] = jnp.zeros_like(out_ref)

    out_ref[...] += jnp.reshape(partial, (1, 1))


def _sort16(x):
    return plsc.sort_key_val(x, x)[0]


def _rv(x):
    return lax.rev(x, (0,))


def _topk_sc_body(ent_hbm, out_hbm, rows_v, res_v, cmin_v):
    wid = lax.axis_index("s") * 2 + lax.axis_index("c")
    base = wid * RPW
    pltpu.sync_copy(ent_hbm.at[pl.ds(N - SCROWS + base, RPW)], rows_v)

    lane = lax.iota(jnp.int32, L)
    inf16 = jnp.full((L,), jnp.inf, jnp.float32)
    NCH = N // L

    def chunk_fn(r, c, carry):
        a, b, bound = carry
        cmin = cmin_v[c]

        def do_merge(_):
            v = rows_v[r, pl.ds(c * L, L)]
            sc = _sort16(v)
            lo1 = _sort16(jnp.minimum(b, _rv(sc)))
            hi1 = _sort16(jnp.maximum(b, _rv(sc)))
            na = _sort16(jnp.minimum(a, _rv(lo1)))
            mid = _sort16(jnp.maximum(a, _rv(lo1)))
            nb = _sort16(jnp.minimum(mid, _rv(hi1)))
            return na, nb, jnp.max(nb)

        return lax.cond(cmin < bound, do_merge, lambda _: carry, 0)

    def row_fn(j, acc):
        r = j

        def premin(c, _):
            cmin_v[c] = jnp.min(rows_v[r, pl.ds(c * L, L)])
            return 0

        lax.fori_loop(0, NCH, premin, 0)

        a, b, _ = lax.fori_loop(0, NCH, functools.partial(chunk_fn, r),
                                (inf16, inf16, jnp.float32(jnp.inf)))
        contrib = a + jnp.where(lane < K - L, b, 0.0)
        s25 = jnp.sum(contrib)
        return jnp.where(lane == j, s25, acc)

    acc = lax.fori_loop(0, RPW, row_fn, jnp.zeros((L,), jnp.float32))
    res_v[pl.ds(0, L)] = acc

    pltpu.sync_copy(res_v, out_hbm.at[wid])


def kernel(distogram_logits, residue_index, asym_id):
    lg = jnp.transpose(distogram_logits, (0, 1, 3, 2)).reshape(N, NBINS, N)
    ri = residue_index.reshape(1, N).astype(jnp.int32)
    ci = asym_id.reshape(1, N).astype(jnp.int32)
    rit = ri.reshape(N, 1)
    cit = ci.reshape(N, 1)

    ent = pl.pallas_call(
        _ent_body,
        grid=(N // RB,),
        in_specs=[
            pl.BlockSpec((RB, NBINS, N), lambda i: (i, 0, 0)),
            pl.BlockSpec((RB, 1), lambda i: (i, 0)),
            pl.BlockSpec((1, N), lambda i: (0, 0)),
            pl.BlockSpec((RB, 1), lambda i: (i, 0)),
            pl.BlockSpec((1, N), lambda i: (0, 0)),
        ],
        out_specs=pl.BlockSpec((RB, N), lambda i: (i, 0)),
        out_shape=jax.ShapeDtypeStruct((N, N), jnp.float32),
    )(lg, rit, ri, cit, ci)

    mesh = plsc.VectorSubcoreMesh(core_axis_name="c", subcore_axis_name="s")
    topk_sc = functools.partial(
        pl.kernel,
        mesh=mesh,
        out_type=jax.ShapeDtypeStruct((NW, L), jnp.float32),
        scratch_types=[
            pltpu.VMEM((RPW, N), jnp.float32),
            pltpu.VMEM((L,), jnp.float32),
            pltpu.SMEM((N // L,), jnp.float32),
        ],
        compiler_params=pltpu.CompilerParams(needs_layout_passes=False),
    )(_topk_sc_body)
    sc_row_sums = topk_sc(ent)

    tc_part = pl.pallas_call(
        _topk_tc_body,
        grid=((N - SCROWS) // TB,),
        in_specs=[pl.BlockSpec((TB, N), lambda i: (i, 0))],
        out_specs=pl.BlockSpec((1, 1), lambda i: (0, 0)),
        out_shape=jax.ShapeDtypeStruct((1, 1), jnp.float32),
    )(ent)

    return tc_part[0, 0] + jnp.sum(sc_row_sums) * (1.0 / (K * N))
